# Initial kernel scaffold; baseline (speedup 1.0000x reference)
import jax
import jax.numpy as jnp
from jax.experimental import pallas as pl


def kernel(x, hg, W1, b1, W2, b2):
    raise NotImplementedError("write your pallas kernel here")


# speedup vs baseline: 2.3027x; 2.3027x over previous
import functools
import jax
import jax.numpy as jnp
from jax import lax
from jax.experimental import pallas as pl
from jax.experimental.pallas import tpu as pltpu
from jax.experimental.pallas import tpu_sc as plsc

N = 10000
NE = 10000
NNZ = 160000
D = 256

TP = 10240
PAD_ROW = 10000
NNZP = 163840
CH = 128
HC = D // 2

NCORE = 2
NSUB = 16
ROWS_PER_TILE = TP // NSUB
RCH = ROWS_PER_TILE // CH
PASS_CPT = NNZP // NSUB // CH

_MESH = plsc.VectorSubcoreMesh(core_axis_name="c", subcore_axis_name="s")


def _zero_rows(buf):
    @pl.loop(0, CH)
    def _(r):
        for c8 in range(HC // 16):
            buf[r, pl.ds(c8 * 16, 16)] = jnp.zeros((16,), jnp.float32)


def _pass_body(gidx, sidx, invdeg, src, out, acc, ig, isb, rows, dbuf, sem,
               relu):
    tid = lax.axis_index("s")

    _zero_rows(rows)
    for m in range(RCH):
        pltpu.sync_copy(rows, acc.at[pl.ds(tid * ROWS_PER_TILE + m * CH, CH)])
    plsc.subcore_barrier()

    @pl.loop(0, PASS_CPT)
    def _(k):
        base = tid * (PASS_CPT * CH) + k * CH
        pltpu.sync_copy(gidx.at[pl.ds(base, CH)], ig)
        pltpu.sync_copy(sidx.at[pl.ds(base, CH)], isb)
        pltpu.async_copy(src.at[ig], rows, sem).wait()
        pltpu.sync_copy(rows, acc.at[isb], add=True)

    plsc.subcore_barrier()

    @pl.loop(0, RCH)
    def _(m):
        rbase = tid * ROWS_PER_TILE + m * CH
        pltpu.sync_copy(acc.at[pl.ds(rbase, CH)], rows)
        pltpu.sync_copy(invdeg.at[pl.ds(rbase, CH)], dbuf)

        @pl.loop(0, CH // 16)
        def _(g):
            dvec = dbuf[pl.ds(g * 16, 16)]
            for j in range(16):
                s = dvec[j]
                r = g * 16 + j
                for c8 in range(HC // 16):
                    v = rows[r, pl.ds(c8 * 16, 16)] * s
                    if relu:
                        v = jnp.maximum(v, 0.0)
                    rows[r, pl.ds(c8 * 16, 16)] = v

        pltpu.sync_copy(rows, out.at[pl.ds(rbase, CH)])


def _make_pass(relu):
    @functools.partial(
        pl.kernel,
        out_type=[
            jax.ShapeDtypeStruct((TP, HC), jnp.float32),
            jax.ShapeDtypeStruct((TP, HC), jnp.float32),
        ],
        mesh=_MESH,
        scratch_types=[
            pltpu.VMEM_SHARED((TP, HC), jnp.float32),
            pltpu.VMEM((CH,), jnp.int32),
            pltpu.VMEM((CH,), jnp.int32),
            pltpu.VMEM((CH, HC), jnp.float32),
            pltpu.VMEM((CH,), jnp.float32),
            pltpu.SemaphoreType.DMA,
        ],
    )
    def k(src0, src1, gidx, sidx, invdeg, out0, out1,
          acc, ig, isb, rows, dbuf, sem):
        cid = lax.axis_index("c")

        @pl.when(cid == 0)
        def _():
            _pass_body(gidx, sidx, invdeg, src0, out0,
                       acc, ig, isb, rows, dbuf, sem, relu)

        @pl.when(cid == 1)
        def _():
            _pass_body(gidx, sidx, invdeg, src1, out1,
                       acc, ig, isb, rows, dbuf, sem, relu)

    return k


_edge_pass = _make_pass(relu=False)
_node_pass = _make_pass(relu=True)


def _deg_body(idx, out, acc, idxb, ones, vbuf):
    tid = lax.axis_index("s")
    base = tid * ROWS_PER_TILE

    @pl.loop(0, ROWS_PER_TILE // 16)
    def _(i):
        vbuf[pl.ds(i * 16, 16)] = jnp.zeros((16,), jnp.float32)
    pltpu.sync_copy(vbuf, acc.at[pl.ds(base, ROWS_PER_TILE)])
    for i in range(CH // 16):
        ones[pl.ds(i * 16, 16)] = jnp.ones((16,), jnp.float32)
    plsc.subcore_barrier()

    @pl.loop(0, PASS_CPT)
    def _(k):
        ebase = tid * (PASS_CPT * CH) + k * CH
        pltpu.sync_copy(idx.at[pl.ds(ebase, CH)], idxb)
        pltpu.sync_copy(ones, acc.at[idxb], add=True)

    plsc.subcore_barrier()

    pltpu.sync_copy(acc.at[pl.ds(base, ROWS_PER_TILE)], vbuf)

    @pl.loop(0, ROWS_PER_TILE // 16)
    def _(i):
        v = vbuf[pl.ds(i * 16, 16)]
        vbuf[pl.ds(i * 16, 16)] = 1.0 / jnp.maximum(v, 1.0)

    pltpu.sync_copy(vbuf, out.at[pl.ds(base, ROWS_PER_TILE)])


@functools.partial(
    pl.kernel,
    out_type=[
        jax.ShapeDtypeStruct((TP,), jnp.float32),
        jax.ShapeDtypeStruct((TP,), jnp.float32),
    ],
    mesh=_MESH,
    scratch_types=[
        pltpu.VMEM_SHARED((TP,), jnp.float32),
        pltpu.VMEM((CH,), jnp.int32),
        pltpu.VMEM((CH,), jnp.float32),
        pltpu.VMEM((ROWS_PER_TILE,), jnp.float32),
    ],
)
def _deg_kernel(eidx, nidx, invde, invdv, acc, idxb, ones, vbuf):
    cid = lax.axis_index("c")

    @pl.when(cid == 0)
    def _():
        _deg_body(eidx, invde, acc, idxb, ones, vbuf)

    @pl.when(cid == 1)
    def _():
        _deg_body(nidx, invdv, acc, idxb, ones, vbuf)


def _mm_body(x_ref, w_ref, b_ref, o_ref):
    o_ref[...] = jnp.dot(x_ref[...], w_ref[...],
                         preferred_element_type=jnp.float32) + b_ref[...]


def _matmul_bias(x, w, b):
    m = x.shape[0]
    blk = 1024
    return pl.pallas_call(
        _mm_body,
        grid=(m // blk,),
        in_specs=[
            pl.BlockSpec((blk, D), lambda i: (i, 0)),
            pl.BlockSpec((D, D), lambda i: (0, 0)),
            pl.BlockSpec((1, D), lambda i: (0, 0)),
        ],
        out_specs=pl.BlockSpec((blk, D), lambda i: (i, 0)),
        out_shape=jax.ShapeDtypeStruct((m, D), jnp.float32),
    )(x, w, b)


@jax.jit
def kernel(x, hg, W1, b1, W2, b2):
    nidx = hg[0].astype(jnp.int32)
    eidx = hg[1].astype(jnp.int32)
    pad = jnp.full((NNZP - NNZ,), PAD_ROW, jnp.int32)
    nidx = jnp.concatenate([nidx, pad])
    eidx = jnp.concatenate([eidx, pad])

    invde, invdv = _deg_kernel(eidx, nidx)

    xp = jnp.pad(x, ((0, TP - N), (0, 0)))
    h = xp
    for (w, b) in ((W1, b1), (W2, b2)):
        xw = _matmul_bias(h, w, b.reshape(1, D))
        e0, e1 = _edge_pass(xw[:, :HC], xw[:, HC:], nidx, eidx, invde)
        h0, h1 = _node_pass(e0, e1, eidx, nidx, invdv)
        h = jnp.concatenate([h0, h1], axis=1)
    return h[:N]


# speedup vs baseline: 2.8906x; 1.2553x over previous
import functools
import jax
import jax.numpy as jnp
from jax import lax
from jax.experimental import pallas as pl
from jax.experimental.pallas import tpu as pltpu
from jax.experimental.pallas import tpu_sc as plsc

N = 10000
NE = 10000
NNZ = 160000
D = 256

TP = 10240
PAD_ROW = 10000
NNZP = 163840
CH = 128
HC = D // 2

NCORE = 2
NSUB = 16
ROWS_PER_TILE = TP // NSUB
ECH = 64
ECPT = NNZP // NSUB // ECH
RCH = ROWS_PER_TILE // ECH
PASS_CPT = NNZP // NSUB // CH

_MESH = plsc.VectorSubcoreMesh(core_axis_name="c", subcore_axis_name="s")


def _zero_rows(buf):
    @pl.loop(0, ECH)
    def _(r):
        for c8 in range(HC // 16):
            buf[r, pl.ds(c8 * 16, 16)] = jnp.zeros((16,), jnp.float32)


def _pass_body(comb, invdeg, src, out, acc, idx_all, rows, dbuf, sems,
               relu):
    tid = lax.axis_index("s")

    pltpu.sync_copy(comb.at[pl.ds(tid * ECPT, ECPT)], idx_all)

    _zero_rows(rows[0])
    for m in range(RCH):
        pltpu.sync_copy(rows[0],
                        acc.at[pl.ds(tid * ROWS_PER_TILE + m * ECH, ECH)])
    plsc.subcore_barrier()

    pltpu.async_copy(src.at[idx_all.at[0, pl.ds(0, ECH)]], rows[0], sems[0])

    @pl.loop(0, ECPT - 2, step=2)
    def _(k0):
        for b in range(2):
            k = k0 + b
            pltpu.make_async_copy(src.at[idx_all.at[k, pl.ds(0, ECH)]], rows[b],
                                  sems[b]).wait()
            pltpu.async_copy(src.at[idx_all.at[k + 1, pl.ds(0, ECH)]], rows[1 - b],
                             sems[1 - b])
            pltpu.sync_copy(rows[b], acc.at[idx_all.at[k, pl.ds(ECH, ECH)]], add=True)

    k = ECPT - 2
    pltpu.make_async_copy(src.at[idx_all.at[k, pl.ds(0, ECH)]], rows[0], sems[0]).wait()
    pltpu.async_copy(src.at[idx_all.at[k + 1, pl.ds(0, ECH)]], rows[1], sems[1])
    pltpu.sync_copy(rows[0], acc.at[idx_all.at[k, pl.ds(ECH, ECH)]], add=True)
    pltpu.make_async_copy(src.at[idx_all.at[k + 1, pl.ds(0, ECH)]], rows[1],
                          sems[1]).wait()
    pltpu.sync_copy(rows[1], acc.at[idx_all.at[k + 1, pl.ds(ECH, ECH)]], add=True)

    plsc.subcore_barrier()

    @pl.loop(0, RCH)
    def _(m):
        rbase = tid * ROWS_PER_TILE + m * ECH
        pltpu.sync_copy(acc.at[pl.ds(rbase, ECH)], rows[0])
        pltpu.sync_copy(invdeg.at[pl.ds(rbase, ECH)], dbuf)

        @pl.loop(0, ECH // 16)
        def _(g):
            dvec = dbuf[pl.ds(g * 16, 16)]
            for j in range(16):
                s = dvec[j]
                r = g * 16 + j
                for c8 in range(HC // 16):
                    v = rows[0][r, pl.ds(c8 * 16, 16)] * s
                    if relu:
                        v = jnp.maximum(v, 0.0)
                    rows[0][r, pl.ds(c8 * 16, 16)] = v

        pltpu.sync_copy(rows[0], out.at[pl.ds(rbase, ECH)])


def _make_pass(relu):
    @functools.partial(
        pl.kernel,
        out_type=[
            jax.ShapeDtypeStruct((TP, HC), jnp.float32),
            jax.ShapeDtypeStruct((TP, HC), jnp.float32),
        ],
        mesh=_MESH,
        scratch_types=[
            pltpu.VMEM_SHARED((TP, HC), jnp.float32),
            pltpu.VMEM((ECPT, 2 * ECH), jnp.int32),
            pltpu.VMEM((ECH, HC), jnp.float32),
            pltpu.VMEM((ECH, HC), jnp.float32),
            pltpu.VMEM((ECH,), jnp.float32),
            pltpu.SemaphoreType.DMA,
            pltpu.SemaphoreType.DMA,
        ],
    )
    def k(src0, src1, comb, invdeg, out0, out1,
          acc, idx_all, rows_a, rows_b, dbuf, sem_a, sem_b):
        cid = lax.axis_index("c")

        @pl.when(cid == 0)
        def _():
            _pass_body(comb, invdeg, src0, out0, acc, idx_all,
                       (rows_a, rows_b), dbuf, (sem_a, sem_b), relu)

        @pl.when(cid == 1)
        def _():
            _pass_body(comb, invdeg, src1, out1, acc, idx_all,
                       (rows_a, rows_b), dbuf, (sem_a, sem_b), relu)

    return k


_edge_pass = _make_pass(relu=False)
_node_pass = _make_pass(relu=True)


def _deg_body(idx, out, acc, idxb, ones, vbuf):
    tid = lax.axis_index("s")
    base = tid * ROWS_PER_TILE

    @pl.loop(0, ROWS_PER_TILE // 16)
    def _(i):
        vbuf[pl.ds(i * 16, 16)] = jnp.zeros((16,), jnp.float32)
    pltpu.sync_copy(vbuf, acc.at[pl.ds(base, ROWS_PER_TILE)])
    for i in range(CH // 16):
        ones[pl.ds(i * 16, 16)] = jnp.ones((16,), jnp.float32)
    plsc.subcore_barrier()

    @pl.loop(0, PASS_CPT)
    def _(k):
        ebase = tid * (PASS_CPT * CH) + k * CH
        pltpu.sync_copy(idx.at[pl.ds(ebase, CH)], idxb)
        pltpu.sync_copy(ones, acc.at[idxb], add=True)

    plsc.subcore_barrier()

    pltpu.sync_copy(acc.at[pl.ds(base, ROWS_PER_TILE)], vbuf)

    @pl.loop(0, ROWS_PER_TILE // 16)
    def _(i):
        v = vbuf[pl.ds(i * 16, 16)]
        vbuf[pl.ds(i * 16, 16)] = 1.0 / jnp.maximum(v, 1.0)

    pltpu.sync_copy(vbuf, out.at[pl.ds(base, ROWS_PER_TILE)])


@functools.partial(
    pl.kernel,
    out_type=[
        jax.ShapeDtypeStruct((TP,), jnp.float32),
        jax.ShapeDtypeStruct((TP,), jnp.float32),
    ],
    mesh=_MESH,
    scratch_types=[
        pltpu.VMEM_SHARED((TP,), jnp.float32),
        pltpu.VMEM((CH,), jnp.int32),
        pltpu.VMEM((CH,), jnp.float32),
        pltpu.VMEM((ROWS_PER_TILE,), jnp.float32),
    ],
)
def _deg_kernel(eidx, nidx, invde, invdv, acc, idxb, ones, vbuf):
    cid = lax.axis_index("c")

    @pl.when(cid == 0)
    def _():
        _deg_body(eidx, invde, acc, idxb, ones, vbuf)

    @pl.when(cid == 1)
    def _():
        _deg_body(nidx, invdv, acc, idxb, ones, vbuf)


def _mm_body(x_ref, w_ref, b_ref, o_ref):
    o_ref[...] = jnp.dot(x_ref[...], w_ref[...],
                         preferred_element_type=jnp.float32) + b_ref[...]


def _matmul_bias(x, w, b):
    m = x.shape[0]
    blk = 1024
    return pl.pallas_call(
        _mm_body,
        grid=(m // blk,),
        in_specs=[
            pl.BlockSpec((blk, D), lambda i: (i, 0)),
            pl.BlockSpec((D, D), lambda i: (0, 0)),
            pl.BlockSpec((1, D), lambda i: (0, 0)),
        ],
        out_specs=pl.BlockSpec((blk, D), lambda i: (i, 0)),
        out_shape=jax.ShapeDtypeStruct((m, D), jnp.float32),
    )(x, w, b)


@jax.jit
def kernel(x, hg, W1, b1, W2, b2):
    nidx = hg[0].astype(jnp.int32)
    eidx = hg[1].astype(jnp.int32)
    pad = jnp.full((NNZP - NNZ,), PAD_ROW, jnp.int32)
    nidx = jnp.concatenate([nidx, pad])
    eidx = jnp.concatenate([eidx, pad])

    invde, invdv = _deg_kernel(eidx, nidx)

    n3 = nidx.reshape(NNZP // ECH, ECH)
    e3 = eidx.reshape(NNZP // ECH, ECH)
    comb_ne = jnp.concatenate([n3, e3], axis=1)
    comb_en = jnp.concatenate([e3, n3], axis=1)

    xp = jnp.pad(x, ((0, TP - N), (0, 0)))
    h = xp
    for (w, b) in ((W1, b1), (W2, b2)):
        xw = _matmul_bias(h, w, b.reshape(1, D))
        e0, e1 = _edge_pass(xw[:, :HC], xw[:, HC:], comb_ne, invde)
        h0, h1 = _node_pass(e0, e1, comb_en, invdv)
        h = jnp.concatenate([h0, h1], axis=1)
    return h[:N]


# speedup vs baseline: 3.4352x; 1.1884x over previous
import functools
import jax
import jax.numpy as jnp
from jax import lax
from jax.experimental import pallas as pl
from jax.experimental.pallas import tpu as pltpu
from jax.experimental.pallas import tpu_sc as plsc

N = 10000
NE = 10000
NNZ = 160000
D = 256

TP = 10240
PAD_ROW = 10000
NNZP = 163840
CH = 128
HC = D // 2

NCORE = 2
NSUB = 16
ROWS_PER_TILE = TP // NSUB
ECH = 64
ECPT = NNZP // NSUB // ECH
RCH = ROWS_PER_TILE // ECH
PASS_CPT = NNZP // NSUB // CH

_MESH = plsc.VectorSubcoreMesh(core_axis_name="c", subcore_axis_name="s")


def _zero_rows(buf):
    @pl.loop(0, ECH)
    def _(r):
        for c8 in range(HC // 16):
            buf[r, pl.ds(c8 * 16, 16)] = jnp.zeros((16,), jnp.float32)


def _pass_body(comb, invdeg, src, out, acc, idx_all, rows, dbuf, sems,
               relu):
    tid = lax.axis_index("s")

    pltpu.sync_copy(comb.at[pl.ds(tid * ECPT, ECPT)], idx_all)

    _zero_rows(rows[0])
    for m in range(RCH):
        pltpu.sync_copy(rows[0],
                        acc.at[pl.ds(tid * ROWS_PER_TILE + m * ECH, ECH)])
    plsc.subcore_barrier()

    gsem, ssem = sems

    def gidx_ref(k):
        return src.at[idx_all.at[k, pl.ds(0, ECH)]]

    def sidx_ref(k):
        return acc.at[idx_all.at[k, pl.ds(ECH, ECH)]]

    def step(k, b, bn, first, issue_gather):
        pltpu.make_async_copy(gidx_ref(k), rows[b], gsem[b]).wait()
        pltpu.async_copy(rows[b], sidx_ref(k), ssem[b], add=True)
        if not first:
            pltpu.make_async_copy(rows[bn], sidx_ref(k - 1), ssem[bn]).wait()
        if issue_gather:
            pltpu.async_copy(gidx_ref(k + 2), rows[bn], gsem[bn])

    pltpu.async_copy(gidx_ref(0), rows[0], gsem[0])
    pltpu.async_copy(gidx_ref(1), rows[1], gsem[1])
    step(0, 0, 2, True, True)

    @pl.loop(1, ECPT - 3, step=3)
    def _(k0):
        for j in range(3):
            b = (1 + j) % 3
            step(k0 + j, b, (b + 2) % 3, False, True)

    step(ECPT - 3, 1, 0, False, True)
    step(ECPT - 2, 2, 1, False, False)
    step(ECPT - 1, 0, 2, False, False)
    pltpu.make_async_copy(rows[0], sidx_ref(ECPT - 1), ssem[0]).wait()

    plsc.subcore_barrier()

    @pl.loop(0, RCH)
    def _(m):
        rbase = tid * ROWS_PER_TILE + m * ECH
        pltpu.sync_copy(acc.at[pl.ds(rbase, ECH)], rows[0])
        pltpu.sync_copy(invdeg.at[pl.ds(rbase, ECH)], dbuf)

        @pl.loop(0, ECH // 16)
        def _(g):
            dvec = dbuf[pl.ds(g * 16, 16)]
            for j in range(16):
                s = dvec[j]
                r = g * 16 + j
                for c8 in range(HC // 16):
                    v = rows[0][r, pl.ds(c8 * 16, 16)] * s
                    if relu:
                        v = jnp.maximum(v, 0.0)
                    rows[0][r, pl.ds(c8 * 16, 16)] = v

        pltpu.sync_copy(rows[0], out.at[pl.ds(rbase, ECH)])


def _make_pass(relu):
    @functools.partial(
        pl.kernel,
        out_type=[
            jax.ShapeDtypeStruct((TP, HC), jnp.float32),
            jax.ShapeDtypeStruct((TP, HC), jnp.float32),
        ],
        mesh=_MESH,
        scratch_types=[
            pltpu.VMEM_SHARED((TP, HC), jnp.float32),
            pltpu.VMEM((ECPT, 2 * ECH), jnp.int32),
            pltpu.VMEM((ECH, HC), jnp.float32),
            pltpu.VMEM((ECH, HC), jnp.float32),
            pltpu.VMEM((ECH, HC), jnp.float32),
            pltpu.VMEM((ECH,), jnp.float32),
            pltpu.SemaphoreType.DMA,
            pltpu.SemaphoreType.DMA,
            pltpu.SemaphoreType.DMA,
            pltpu.SemaphoreType.DMA,
            pltpu.SemaphoreType.DMA,
            pltpu.SemaphoreType.DMA,
        ],
    )
    def k(src0, src1, comb, invdeg, out0, out1,
          acc, idx_all, rows_a, rows_b, rows_c, dbuf,
          g0, g1, g2, s0, s1, s2):
        cid = lax.axis_index("c")
        rows = (rows_a, rows_b, rows_c)
        sems = ((g0, g1, g2), (s0, s1, s2))

        @pl.when(cid == 0)
        def _():
            _pass_body(comb, invdeg, src0, out0, acc, idx_all,
                       rows, dbuf, sems, relu)

        @pl.when(cid == 1)
        def _():
            _pass_body(comb, invdeg, src1, out1, acc, idx_all,
                       rows, dbuf, sems, relu)

    return k


_edge_pass = _make_pass(relu=False)
_node_pass = _make_pass(relu=True)


def _deg_body(idx, out, acc, idxb, ones, vbuf):
    tid = lax.axis_index("s")
    base = tid * ROWS_PER_TILE

    @pl.loop(0, ROWS_PER_TILE // 16)
    def _(i):
        vbuf[pl.ds(i * 16, 16)] = jnp.zeros((16,), jnp.float32)
    pltpu.sync_copy(vbuf, acc.at[pl.ds(base, ROWS_PER_TILE)])
    for i in range(CH // 16):
        ones[pl.ds(i * 16, 16)] = jnp.ones((16,), jnp.float32)
    plsc.subcore_barrier()

    @pl.loop(0, PASS_CPT)
    def _(k):
        ebase = tid * (PASS_CPT * CH) + k * CH
        pltpu.sync_copy(idx.at[pl.ds(ebase, CH)], idxb)
        pltpu.sync_copy(ones, acc.at[idxb], add=True)

    plsc.subcore_barrier()

    pltpu.sync_copy(acc.at[pl.ds(base, ROWS_PER_TILE)], vbuf)

    @pl.loop(0, ROWS_PER_TILE // 16)
    def _(i):
        v = vbuf[pl.ds(i * 16, 16)]
        vbuf[pl.ds(i * 16, 16)] = 1.0 / jnp.maximum(v, 1.0)

    pltpu.sync_copy(vbuf, out.at[pl.ds(base, ROWS_PER_TILE)])


@functools.partial(
    pl.kernel,
    out_type=[
        jax.ShapeDtypeStruct((TP,), jnp.float32),
        jax.ShapeDtypeStruct((TP,), jnp.float32),
    ],
    mesh=_MESH,
    scratch_types=[
        pltpu.VMEM_SHARED((TP,), jnp.float32),
        pltpu.VMEM((CH,), jnp.int32),
        pltpu.VMEM((CH,), jnp.float32),
        pltpu.VMEM((ROWS_PER_TILE,), jnp.float32),
    ],
)
def _deg_kernel(eidx, nidx, invde, invdv, acc, idxb, ones, vbuf):
    cid = lax.axis_index("c")

    @pl.when(cid == 0)
    def _():
        _deg_body(eidx, invde, acc, idxb, ones, vbuf)

    @pl.when(cid == 1)
    def _():
        _deg_body(nidx, invdv, acc, idxb, ones, vbuf)


def _mm_body(x_ref, w_ref, b_ref, o_ref):
    o_ref[...] = jnp.dot(x_ref[...], w_ref[...],
                         preferred_element_type=jnp.float32) + b_ref[...]


def _matmul_bias(x, w, b):
    m = x.shape[0]
    blk = 1024
    return pl.pallas_call(
        _mm_body,
        grid=(m // blk,),
        in_specs=[
            pl.BlockSpec((blk, D), lambda i: (i, 0)),
            pl.BlockSpec((D, D), lambda i: (0, 0)),
            pl.BlockSpec((1, D), lambda i: (0, 0)),
        ],
        out_specs=pl.BlockSpec((blk, D), lambda i: (i, 0)),
        out_shape=jax.ShapeDtypeStruct((m, D), jnp.float32),
    )(x, w, b)


@jax.jit
def kernel(x, hg, W1, b1, W2, b2):
    nidx = hg[0].astype(jnp.int32)
    eidx = hg[1].astype(jnp.int32)
    pad = jnp.full((NNZP - NNZ,), PAD_ROW, jnp.int32)
    nidx = jnp.concatenate([nidx, pad])
    eidx = jnp.concatenate([eidx, pad])

    invde, invdv = _deg_kernel(eidx, nidx)

    n3 = nidx.reshape(NNZP // ECH, ECH)
    e3 = eidx.reshape(NNZP // ECH, ECH)
    comb_ne = jnp.concatenate([n3, e3], axis=1)
    comb_en = jnp.concatenate([e3, n3], axis=1)

    xp = jnp.pad(x, ((0, TP - N), (0, 0)))
    h = xp
    for (w, b) in ((W1, b1), (W2, b2)):
        xw = _matmul_bias(h, w, b.reshape(1, D))
        e0, e1 = _edge_pass(xw[:, :HC], xw[:, HC:], comb_ne, invde)
        h0, h1 = _node_pass(e0, e1, comb_en, invdv)
        h = jnp.concatenate([h0, h1], axis=1)
    return h[:N]


# speedup vs baseline: 3.5497x; 1.0333x over previous
import functools
import jax
import jax.numpy as jnp
from jax import lax
from jax.experimental import pallas as pl
from jax.experimental.pallas import tpu as pltpu
from jax.experimental.pallas import tpu_sc as plsc

N = 10000
NE = 10000
NNZ = 160000
D = 256

TP = 10240
PAD_ROW = 10000
NNZP = 163840
CH = 128
HC = D // 2

NCORE = 2
NSUB = 16
ROWS_PER_TILE = TP // NSUB
ECH = 64
ECPT = NNZP // NSUB // ECH
RCH = ROWS_PER_TILE // ECH
PASS_CPT = NNZP // NSUB // CH

_MESH = plsc.VectorSubcoreMesh(core_axis_name="c", subcore_axis_name="s")


def _zero_rows(buf):
    @pl.loop(0, ECH)
    def _(r):
        for c8 in range(HC // 16):
            buf[r, pl.ds(c8 * 16, 16)] = jnp.zeros((16,), jnp.float32)


def _pass_body(comb, invdeg, src, out, acc, idx_all, rows, dbuf, sems,
               relu):
    tid = lax.axis_index("s")

    pltpu.sync_copy(comb.at[pl.ds(tid * ECPT, ECPT)], idx_all)

    _zero_rows(rows[0])
    for m in range(RCH):
        pltpu.sync_copy(rows[0],
                        acc.at[pl.ds(tid * ROWS_PER_TILE + m * ECH, ECH)])
    plsc.subcore_barrier()

    gsem, ssem = sems

    def gidx_ref(k):
        return src.at[idx_all.at[k, pl.ds(0, ECH)]]

    def sidx_ref(k):
        return acc.at[idx_all.at[k, pl.ds(ECH, ECH)]]

    def step(k, b, bn, first, issue_gather):
        pltpu.make_async_copy(gidx_ref(k), rows[b], gsem[b]).wait()
        pltpu.async_copy(rows[b], sidx_ref(k), ssem[b], add=True)
        if not first:
            pltpu.make_async_copy(rows[bn], sidx_ref(k - 1), ssem[bn]).wait()
        if issue_gather:
            pltpu.async_copy(gidx_ref(k + 2), rows[bn], gsem[bn])

    pltpu.async_copy(gidx_ref(0), rows[0], gsem[0])
    pltpu.async_copy(gidx_ref(1), rows[1], gsem[1])
    step(0, 0, 2, True, True)

    @pl.loop(1, ECPT - 3, step=3)
    def _(k0):
        for j in range(3):
            b = (1 + j) % 3
            step(k0 + j, b, (b + 2) % 3, False, True)

    step(ECPT - 3, 1, 0, False, True)
    step(ECPT - 2, 2, 1, False, False)
    step(ECPT - 1, 0, 2, False, False)
    pltpu.make_async_copy(rows[0], sidx_ref(ECPT - 1), ssem[0]).wait()

    plsc.subcore_barrier()

    def _norm_chunk(m, b):
        rbase = tid * ROWS_PER_TILE + m * ECH
        pltpu.sync_copy(acc.at[pl.ds(rbase, ECH)], rows[b])
        pltpu.sync_copy(invdeg.at[pl.ds(rbase, ECH)], dbuf)

        @pl.loop(0, ECH // 16)
        def _(g):
            dvec = dbuf[pl.ds(g * 16, 16)]
            for j in range(16):
                s = dvec[j]
                r = g * 16 + j
                for c8 in range(HC // 16):
                    v = rows[b][r, pl.ds(c8 * 16, 16)] * s
                    if relu:
                        v = jnp.maximum(v, 0.0)
                    rows[b][r, pl.ds(c8 * 16, 16)] = v

        pltpu.async_copy(rows[b], out.at[pl.ds(rbase, ECH)], ssem[b])

    def _norm_wait(m, b):
        rbase = tid * ROWS_PER_TILE + m * ECH
        pltpu.make_async_copy(rows[b], out.at[pl.ds(rbase, ECH)],
                              ssem[b]).wait()

    _norm_chunk(0, 0)
    _norm_chunk(1, 1)

    @pl.loop(2, RCH, step=2)
    def _(m0):
        for b in range(2):
            m = m0 + b
            _norm_wait(m - 2, b)
            _norm_chunk(m, b)

    _norm_wait(RCH - 2, 0)
    _norm_wait(RCH - 1, 1)


def _make_pass(relu):
    @functools.partial(
        pl.kernel,
        out_type=[
            jax.ShapeDtypeStruct((TP, HC), jnp.float32),
            jax.ShapeDtypeStruct((TP, HC), jnp.float32),
        ],
        mesh=_MESH,
        scratch_types=[
            pltpu.VMEM_SHARED((TP, HC), jnp.float32),
            pltpu.VMEM((ECPT, 2 * ECH), jnp.int32),
            pltpu.VMEM((ECH, HC), jnp.float32),
            pltpu.VMEM((ECH, HC), jnp.float32),
            pltpu.VMEM((ECH, HC), jnp.float32),
            pltpu.VMEM((ECH,), jnp.float32),
            pltpu.SemaphoreType.DMA,
            pltpu.SemaphoreType.DMA,
            pltpu.SemaphoreType.DMA,
            pltpu.SemaphoreType.DMA,
            pltpu.SemaphoreType.DMA,
            pltpu.SemaphoreType.DMA,
        ],
    )
    def k(src0, src1, comb, invdeg, out0, out1,
          acc, idx_all, rows_a, rows_b, rows_c, dbuf,
          g0, g1, g2, s0, s1, s2):
        cid = lax.axis_index("c")
        rows = (rows_a, rows_b, rows_c)
        sems = ((g0, g1, g2), (s0, s1, s2))

        @pl.when(cid == 0)
        def _():
            _pass_body(comb, invdeg, src0, out0, acc, idx_all,
                       rows, dbuf, sems, relu)

        @pl.when(cid == 1)
        def _():
            _pass_body(comb, invdeg, src1, out1, acc, idx_all,
                       rows, dbuf, sems, relu)

    return k


_edge_pass = _make_pass(relu=False)
_node_pass = _make_pass(relu=True)


def _deg_body(idx, out, acc, idx_all, ones, vbuf, sem):
    tid = lax.axis_index("s")
    base = tid * ROWS_PER_TILE

    pltpu.async_copy(idx.at[pl.ds(tid * PASS_CPT, PASS_CPT)], idx_all,
                     sem[0])

    @pl.loop(0, ROWS_PER_TILE // 16)
    def _(i):
        vbuf[pl.ds(i * 16, 16)] = jnp.zeros((16,), jnp.float32)
    pltpu.sync_copy(vbuf, acc.at[pl.ds(base, ROWS_PER_TILE)])
    for i in range(CH // 16):
        ones[pl.ds(i * 16, 16)] = jnp.ones((16,), jnp.float32)
    pltpu.make_async_copy(idx.at[pl.ds(tid * PASS_CPT, PASS_CPT)], idx_all,
                          sem[0]).wait()
    plsc.subcore_barrier()

    def sc(k, s):
        pltpu.async_copy(ones, acc.at[idx_all.at[k]], s, add=True)

    def wt(k, s):
        pltpu.make_async_copy(ones, acc.at[idx_all.at[k]], s).wait()

    sc(0, sem[0])
    sc(1, sem[1])

    @pl.loop(2, PASS_CPT, step=2)
    def _(k0):
        wt(k0 - 2, sem[0])
        sc(k0, sem[0])
        wt(k0 - 1, sem[1])
        sc(k0 + 1, sem[1])

    wt(PASS_CPT - 2, sem[0])
    wt(PASS_CPT - 1, sem[1])
    plsc.subcore_barrier()

    pltpu.sync_copy(acc.at[pl.ds(base, ROWS_PER_TILE)], vbuf)

    @pl.loop(0, ROWS_PER_TILE // 16)
    def _(i):
        v = vbuf[pl.ds(i * 16, 16)]
        vbuf[pl.ds(i * 16, 16)] = 1.0 / jnp.maximum(v, 1.0)

    pltpu.sync_copy(vbuf, out.at[pl.ds(base, ROWS_PER_TILE)])


@functools.partial(
    pl.kernel,
    out_type=[
        jax.ShapeDtypeStruct((TP,), jnp.float32),
        jax.ShapeDtypeStruct((TP,), jnp.float32),
    ],
    mesh=_MESH,
    scratch_types=[
        pltpu.VMEM_SHARED((TP,), jnp.float32),
        pltpu.VMEM((PASS_CPT, CH), jnp.int32),
        pltpu.VMEM((CH,), jnp.float32),
        pltpu.VMEM((ROWS_PER_TILE,), jnp.float32),
        pltpu.SemaphoreType.DMA,
        pltpu.SemaphoreType.DMA,
    ],
)
def _deg_kernel(eidx, nidx, invde, invdv, acc, idx_all, ones, vbuf,
                sem_a, sem_b):
    sem = (sem_a, sem_b)
    cid = lax.axis_index("c")

    @pl.when(cid == 0)
    def _():
        _deg_body(eidx, invde, acc, idx_all, ones, vbuf, sem)

    @pl.when(cid == 1)
    def _():
        _deg_body(nidx, invdv, acc, idx_all, ones, vbuf, sem)


def _mm_body(x_ref, w_ref, b_ref, o_ref):
    o_ref[...] = jnp.dot(x_ref[...], w_ref[...],
                         preferred_element_type=jnp.float32) + b_ref[...]


def _matmul_bias(x, w, b):
    m = x.shape[0]
    blk = 1024
    return pl.pallas_call(
        _mm_body,
        grid=(m // blk,),
        in_specs=[
            pl.BlockSpec((blk, D), lambda i: (i, 0)),
            pl.BlockSpec((D, D), lambda i: (0, 0)),
            pl.BlockSpec((1, D), lambda i: (0, 0)),
        ],
        out_specs=pl.BlockSpec((blk, D), lambda i: (i, 0)),
        out_shape=jax.ShapeDtypeStruct((m, D), jnp.float32),
    )(x, w, b)


@jax.jit
def kernel(x, hg, W1, b1, W2, b2):
    nidx = hg[0].astype(jnp.int32)
    eidx = hg[1].astype(jnp.int32)
    pad = jnp.full((NNZP - NNZ,), PAD_ROW, jnp.int32)
    nidx = jnp.concatenate([nidx, pad])
    eidx = jnp.concatenate([eidx, pad])

    invde, invdv = _deg_kernel(eidx.reshape(NNZP // CH, CH),
                               nidx.reshape(NNZP // CH, CH))

    n3 = nidx.reshape(NNZP // ECH, ECH)
    e3 = eidx.reshape(NNZP // ECH, ECH)
    comb_ne = jnp.concatenate([n3, e3], axis=1)
    comb_en = jnp.concatenate([e3, n3], axis=1)

    xp = jnp.pad(x, ((0, TP - N), (0, 0)))
    h = xp
    for (w, b) in ((W1, b1), (W2, b2)):
        xw = _matmul_bias(h, w, b.reshape(1, D))
        e0, e1 = _edge_pass(xw[:, :HC], xw[:, HC:], comb_ne, invde)
        h0, h1 = _node_pass(e0, e1, comb_en, invdv)
        h = jnp.concatenate([h0, h1], axis=1)
    return h[:N]


# speedup vs baseline: 3.5735x; 1.0067x over previous
import functools
import jax
import jax.numpy as jnp
from jax import lax
from jax.experimental import pallas as pl
from jax.experimental.pallas import tpu as pltpu
from jax.experimental.pallas import tpu_sc as plsc

N = 10000
NE = 10000
NNZ = 160000
D = 256

TP = 10240
PAD_ROW = 10000
NNZP = 163840
CH = 128
HC = D // 2

NCORE = 2
NSUB = 16
ROWS_PER_TILE = TP // NSUB
ECH = 64
ECPT = NNZP // NSUB // ECH
RCH = ROWS_PER_TILE // ECH
PASS_CPT = NNZP // NSUB // CH

_MESH = plsc.VectorSubcoreMesh(core_axis_name="c", subcore_axis_name="s")


def _zero_rows(buf):
    @pl.loop(0, ECH)
    def _(r):
        for c8 in range(HC // 16):
            buf[r, pl.ds(c8 * 16, 16)] = jnp.zeros((16,), jnp.float32)


def _pass_body(comb, invdeg, src, out, acc, idx_all, rows, dbuf, sems,
               relu):
    tid = lax.axis_index("s")

    pltpu.sync_copy(comb.at[pl.ds(tid * ECPT, ECPT)], idx_all)

    _zero_rows(rows[0])
    for m in range(RCH):
        pltpu.sync_copy(rows[0],
                        acc.at[pl.ds(tid * ROWS_PER_TILE + m * ECH, ECH)])
    plsc.subcore_barrier()

    gsem, ssem = sems

    def gidx_ref(k):
        return src.at[idx_all.at[k, pl.ds(0, ECH)]]

    def sidx_ref(k):
        return acc.at[idx_all.at[k, pl.ds(ECH, ECH)]]

    def step(k, b, bn, first, issue_gather):
        if not first:
            pltpu.make_async_copy(rows[bn], sidx_ref(k - 1), ssem[bn]).wait()
        if issue_gather:
            pltpu.async_copy(gidx_ref(k + 2), rows[bn], gsem[bn])
        pltpu.make_async_copy(gidx_ref(k), rows[b], gsem[b]).wait()
        pltpu.async_copy(rows[b], sidx_ref(k), ssem[b], add=True)

    pltpu.async_copy(gidx_ref(0), rows[0], gsem[0])
    pltpu.async_copy(gidx_ref(1), rows[1], gsem[1])
    step(0, 0, 2, True, True)

    @pl.loop(1, ECPT - 3, step=3)
    def _(k0):
        for j in range(3):
            b = (1 + j) % 3
            step(k0 + j, b, (b + 2) % 3, False, True)

    step(ECPT - 3, 1, 0, False, True)
    step(ECPT - 2, 2, 1, False, False)
    step(ECPT - 1, 0, 2, False, False)
    pltpu.make_async_copy(rows[0], sidx_ref(ECPT - 1), ssem[0]).wait()

    plsc.subcore_barrier()

    def _norm_chunk(m, b):
        rbase = tid * ROWS_PER_TILE + m * ECH
        pltpu.sync_copy(acc.at[pl.ds(rbase, ECH)], rows[b])
        pltpu.sync_copy(invdeg.at[pl.ds(rbase, ECH)], dbuf)

        @pl.loop(0, ECH // 16)
        def _(g):
            dvec = dbuf[pl.ds(g * 16, 16)]
            for j in range(16):
                s = dvec[j]
                r = g * 16 + j
                for c8 in range(HC // 16):
                    v = rows[b][r, pl.ds(c8 * 16, 16)] * s
                    if relu:
                        v = jnp.maximum(v, 0.0)
                    rows[b][r, pl.ds(c8 * 16, 16)] = v

        pltpu.async_copy(rows[b], out.at[pl.ds(rbase, ECH)], ssem[b])

    def _norm_wait(m, b):
        rbase = tid * ROWS_PER_TILE + m * ECH
        pltpu.make_async_copy(rows[b], out.at[pl.ds(rbase, ECH)],
                              ssem[b]).wait()

    _norm_chunk(0, 0)
    _norm_chunk(1, 1)

    @pl.loop(2, RCH, step=2)
    def _(m0):
        for b in range(2):
            m = m0 + b
            _norm_wait(m - 2, b)
            _norm_chunk(m, b)

    _norm_wait(RCH - 2, 0)
    _norm_wait(RCH - 1, 1)


def _make_pass(relu):
    @functools.partial(
        pl.kernel,
        out_type=[
            jax.ShapeDtypeStruct((TP, HC), jnp.float32),
            jax.ShapeDtypeStruct((TP, HC), jnp.float32),
        ],
        mesh=_MESH,
        scratch_types=[
            pltpu.VMEM_SHARED((TP, HC), jnp.float32),
            pltpu.VMEM((ECPT, 2 * ECH), jnp.int32),
            pltpu.VMEM((ECH, HC), jnp.float32),
            pltpu.VMEM((ECH, HC), jnp.float32),
            pltpu.VMEM((ECH, HC), jnp.float32),
            pltpu.VMEM((ECH,), jnp.float32),
            pltpu.SemaphoreType.DMA,
            pltpu.SemaphoreType.DMA,
            pltpu.SemaphoreType.DMA,
            pltpu.SemaphoreType.DMA,
            pltpu.SemaphoreType.DMA,
            pltpu.SemaphoreType.DMA,
        ],
    )
    def k(src0, src1, comb, invdeg, out0, out1,
          acc, idx_all, rows_a, rows_b, rows_c, dbuf,
          g0, g1, g2, s0, s1, s2):
        cid = lax.axis_index("c")
        rows = (rows_a, rows_b, rows_c)
        sems = ((g0, g1, g2), (s0, s1, s2))

        @pl.when(cid == 0)
        def _():
            _pass_body(comb, invdeg, src0, out0, acc, idx_all,
                       rows, dbuf, sems, relu)

        @pl.when(cid == 1)
        def _():
            _pass_body(comb, invdeg, src1, out1, acc, idx_all,
                       rows, dbuf, sems, relu)

    return k


_edge_pass = _make_pass(relu=False)
_node_pass = _make_pass(relu=True)


def _deg_body(idx, out, acc, idx_all, ones, vbuf, sem):
    tid = lax.axis_index("s")
    base = tid * ROWS_PER_TILE

    pltpu.async_copy(idx.at[pl.ds(tid * PASS_CPT, PASS_CPT)], idx_all,
                     sem[0])

    @pl.loop(0, ROWS_PER_TILE // 16)
    def _(i):
        vbuf[pl.ds(i * 16, 16)] = jnp.zeros((16,), jnp.float32)
    pltpu.sync_copy(vbuf, acc.at[pl.ds(base, ROWS_PER_TILE)])
    for i in range(CH // 16):
        ones[pl.ds(i * 16, 16)] = jnp.ones((16,), jnp.float32)
    pltpu.make_async_copy(idx.at[pl.ds(tid * PASS_CPT, PASS_CPT)], idx_all,
                          sem[0]).wait()
    plsc.subcore_barrier()

    def sc(k, s):
        pltpu.async_copy(ones, acc.at[idx_all.at[k]], s, add=True)

    def wt(k, s):
        pltpu.make_async_copy(ones, acc.at[idx_all.at[k]], s).wait()

    sc(0, sem[0])
    sc(1, sem[1])

    @pl.loop(2, PASS_CPT, step=2)
    def _(k0):
        wt(k0 - 2, sem[0])
        sc(k0, sem[0])
        wt(k0 - 1, sem[1])
        sc(k0 + 1, sem[1])

    wt(PASS_CPT - 2, sem[0])
    wt(PASS_CPT - 1, sem[1])
    plsc.subcore_barrier()

    pltpu.sync_copy(acc.at[pl.ds(base, ROWS_PER_TILE)], vbuf)

    @pl.loop(0, ROWS_PER_TILE // 16)
    def _(i):
        v = vbuf[pl.ds(i * 16, 16)]
        vbuf[pl.ds(i * 16, 16)] = 1.0 / jnp.maximum(v, 1.0)

    pltpu.sync_copy(vbuf, out.at[pl.ds(base, ROWS_PER_TILE)])


@functools.partial(
    pl.kernel,
    out_type=[
        jax.ShapeDtypeStruct((TP,), jnp.float32),
        jax.ShapeDtypeStruct((TP,), jnp.float32),
    ],
    mesh=_MESH,
    scratch_types=[
        pltpu.VMEM_SHARED((TP,), jnp.float32),
        pltpu.VMEM((PASS_CPT, CH), jnp.int32),
        pltpu.VMEM((CH,), jnp.float32),
        pltpu.VMEM((ROWS_PER_TILE,), jnp.float32),
        pltpu.SemaphoreType.DMA,
        pltpu.SemaphoreType.DMA,
    ],
)
def _deg_kernel(eidx, nidx, invde, invdv, acc, idx_all, ones, vbuf,
                sem_a, sem_b):
    sem = (sem_a, sem_b)
    cid = lax.axis_index("c")

    @pl.when(cid == 0)
    def _():
        _deg_body(eidx, invde, acc, idx_all, ones, vbuf, sem)

    @pl.when(cid == 1)
    def _():
        _deg_body(nidx, invdv, acc, idx_all, ones, vbuf, sem)


def _mm_body(x_ref, w_ref, b_ref, o_ref):
    o_ref[...] = jnp.dot(x_ref[...], w_ref[...],
                         preferred_element_type=jnp.float32) + b_ref[...]


def _matmul_bias(x, w, b):
    m = x.shape[0]
    blk = 1024
    return pl.pallas_call(
        _mm_body,
        grid=(m // blk,),
        in_specs=[
            pl.BlockSpec((blk, D), lambda i: (i, 0)),
            pl.BlockSpec((D, D), lambda i: (0, 0)),
            pl.BlockSpec((1, D), lambda i: (0, 0)),
        ],
        out_specs=pl.BlockSpec((blk, D), lambda i: (i, 0)),
        out_shape=jax.ShapeDtypeStruct((m, D), jnp.float32),
    )(x, w, b)


@jax.jit
def kernel(x, hg, W1, b1, W2, b2):
    nidx = hg[0].astype(jnp.int32)
    eidx = hg[1].astype(jnp.int32)
    pad = jnp.full((NNZP - NNZ,), PAD_ROW, jnp.int32)
    nidx = jnp.concatenate([nidx, pad])
    eidx = jnp.concatenate([eidx, pad])

    invde, invdv = _deg_kernel(eidx.reshape(NNZP // CH, CH),
                               nidx.reshape(NNZP // CH, CH))

    n3 = nidx.reshape(NNZP // ECH, ECH)
    e3 = eidx.reshape(NNZP // ECH, ECH)
    comb_ne = jnp.concatenate([n3, e3], axis=1)
    comb_en = jnp.concatenate([e3, n3], axis=1)

    xp = jnp.pad(x, ((0, TP - N), (0, 0)))
    h = xp
    for (w, b) in ((W1, b1), (W2, b2)):
        xw = _matmul_bias(h, w, b.reshape(1, D))
        e0, e1 = _edge_pass(xw[:, :HC], xw[:, HC:], comb_ne, invde)
        h0, h1 = _node_pass(e0, e1, comb_en, invdv)
        h = jnp.concatenate([h0, h1], axis=1)
    return h[:N]


# speedup vs baseline: 3.5954x; 1.0061x over previous
import functools
import jax
import jax.numpy as jnp
from jax import lax
from jax.experimental import pallas as pl
from jax.experimental.pallas import tpu as pltpu
from jax.experimental.pallas import tpu_sc as plsc

N = 10000
NE = 10000
NNZ = 160000
D = 256

TP = 10240
PAD_ROW = 10000
NNZP = 163840
CH = 128
HC = D // 2

NCORE = 2
NSUB = 16
ROWS_PER_TILE = TP // NSUB
ECH = 64
ECPT = NNZP // NSUB // ECH
RCH = ROWS_PER_TILE // ECH
PASS_CPT = NNZP // NSUB // CH

_MESH = plsc.VectorSubcoreMesh(core_axis_name="c", subcore_axis_name="s")


def _zero_rows(buf):
    @pl.loop(0, ECH)
    def _(r):
        for c8 in range(HC // 16):
            buf[r, pl.ds(c8 * 16, 16)] = jnp.zeros((16,), jnp.float32)


def _pass_body(comb, invdeg, src, out, acc, idx_all, rows, dbuf, sems,
               relu):
    tid = lax.axis_index("s")

    gsem, ssem = sems
    pltpu.async_copy(comb.at[pl.ds(tid * ECPT, ECPT)], idx_all, gsem[2])

    _zero_rows(rows[0])
    for m in range(RCH):
        pltpu.sync_copy(rows[0],
                        acc.at[pl.ds(tid * ROWS_PER_TILE + m * ECH, ECH)])
    pltpu.make_async_copy(comb.at[pl.ds(tid * ECPT, ECPT)], idx_all,
                          gsem[2]).wait()
    plsc.subcore_barrier()

    def gidx_ref(k):
        return src.at[idx_all.at[k, pl.ds(0, ECH)]]

    def sidx_ref(k):
        return acc.at[idx_all.at[k, pl.ds(ECH, ECH)]]

    def step(k, b, bn, first, issue_gather):
        if not first:
            pltpu.make_async_copy(rows[bn], sidx_ref(k - 1), ssem[bn]).wait()
        if issue_gather:
            pltpu.async_copy(gidx_ref(k + 2), rows[bn], gsem[bn])
        pltpu.make_async_copy(gidx_ref(k), rows[b], gsem[b]).wait()
        pltpu.async_copy(rows[b], sidx_ref(k), ssem[b], add=True)

    pltpu.async_copy(gidx_ref(0), rows[0], gsem[0])
    pltpu.async_copy(gidx_ref(1), rows[1], gsem[1])
    step(0, 0, 2, True, True)

    @pl.loop(1, ECPT - 3, step=3)
    def _(k0):
        for j in range(3):
            b = (1 + j) % 3
            step(k0 + j, b, (b + 2) % 3, False, True)

    step(ECPT - 3, 1, 0, False, True)
    step(ECPT - 2, 2, 1, False, False)
    step(ECPT - 1, 0, 2, False, False)
    pltpu.make_async_copy(rows[0], sidx_ref(ECPT - 1), ssem[0]).wait()

    plsc.subcore_barrier()

    def _norm_chunk(m, b):
        rbase = tid * ROWS_PER_TILE + m * ECH
        pltpu.sync_copy(acc.at[pl.ds(rbase, ECH)], rows[b])
        pltpu.sync_copy(invdeg.at[pl.ds(rbase, ECH)], dbuf)

        @pl.loop(0, ECH // 16)
        def _(g):
            dvec = dbuf[pl.ds(g * 16, 16)]
            for j in range(16):
                s = dvec[j]
                r = g * 16 + j
                for c8 in range(HC // 16):
                    v = rows[b][r, pl.ds(c8 * 16, 16)] * s
                    if relu:
                        v = jnp.maximum(v, 0.0)
                    rows[b][r, pl.ds(c8 * 16, 16)] = v

        pltpu.async_copy(rows[b], out.at[pl.ds(rbase, ECH)], ssem[b])

    def _norm_wait(m, b):
        rbase = tid * ROWS_PER_TILE + m * ECH
        pltpu.make_async_copy(rows[b], out.at[pl.ds(rbase, ECH)],
                              ssem[b]).wait()

    _norm_chunk(0, 0)
    _norm_chunk(1, 1)

    @pl.loop(2, RCH, step=2)
    def _(m0):
        for b in range(2):
            m = m0 + b
            _norm_wait(m - 2, b)
            _norm_chunk(m, b)

    _norm_wait(RCH - 2, 0)
    _norm_wait(RCH - 1, 1)


def _make_pass(relu):
    @functools.partial(
        pl.kernel,
        out_type=[
            jax.ShapeDtypeStruct((TP, HC), jnp.float32),
            jax.ShapeDtypeStruct((TP, HC), jnp.float32),
        ],
        mesh=_MESH,
        scratch_types=[
            pltpu.VMEM_SHARED((TP, HC), jnp.float32),
            pltpu.VMEM((ECPT, 2 * ECH), jnp.int32),
            pltpu.VMEM((ECH, HC), jnp.float32),
            pltpu.VMEM((ECH, HC), jnp.float32),
            pltpu.VMEM((ECH, HC), jnp.float32),
            pltpu.VMEM((ECH,), jnp.float32),
            pltpu.SemaphoreType.DMA,
            pltpu.SemaphoreType.DMA,
            pltpu.SemaphoreType.DMA,
            pltpu.SemaphoreType.DMA,
            pltpu.SemaphoreType.DMA,
            pltpu.SemaphoreType.DMA,
        ],
    )
    def k(src0, src1, comb, invdeg, out0, out1,
          acc, idx_all, rows_a, rows_b, rows_c, dbuf,
          g0, g1, g2, s0, s1, s2):
        cid = lax.axis_index("c")
        rows = (rows_a, rows_b, rows_c)
        sems = ((g0, g1, g2), (s0, s1, s2))

        @pl.when(cid == 0)
        def _():
            _pass_body(comb, invdeg, src0, out0, acc, idx_all,
                       rows, dbuf, sems, relu)

        @pl.when(cid == 1)
        def _():
            _pass_body(comb, invdeg, src1, out1, acc, idx_all,
                       rows, dbuf, sems, relu)

    return k


_edge_pass = _make_pass(relu=False)
_node_pass = _make_pass(relu=True)


def _deg_body(idx, out, acc, idx_all, ones, vbuf, sem):
    tid = lax.axis_index("s")
    base = tid * ROWS_PER_TILE

    pltpu.async_copy(idx.at[pl.ds(tid * PASS_CPT, PASS_CPT)], idx_all,
                     sem[0])

    @pl.loop(0, ROWS_PER_TILE // 16)
    def _(i):
        vbuf[pl.ds(i * 16, 16)] = jnp.zeros((16,), jnp.float32)
    pltpu.sync_copy(vbuf, acc.at[pl.ds(base, ROWS_PER_TILE)])
    for i in range(CH // 16):
        ones[pl.ds(i * 16, 16)] = jnp.ones((16,), jnp.float32)
    pltpu.make_async_copy(idx.at[pl.ds(tid * PASS_CPT, PASS_CPT)], idx_all,
                          sem[0]).wait()
    plsc.subcore_barrier()

    def sc(k, s):
        pltpu.async_copy(ones, acc.at[idx_all.at[k]], s, add=True)

    def wt(k, s):
        pltpu.make_async_copy(ones, acc.at[idx_all.at[k]], s).wait()

    sc(0, sem[0])
    sc(1, sem[1])

    @pl.loop(2, PASS_CPT, step=2)
    def _(k0):
        wt(k0 - 2, sem[0])
        sc(k0, sem[0])
        wt(k0 - 1, sem[1])
        sc(k0 + 1, sem[1])

    wt(PASS_CPT - 2, sem[0])
    wt(PASS_CPT - 1, sem[1])
    plsc.subcore_barrier()

    pltpu.sync_copy(acc.at[pl.ds(base, ROWS_PER_TILE)], vbuf)

    @pl.loop(0, ROWS_PER_TILE // 16)
    def _(i):
        v = vbuf[pl.ds(i * 16, 16)]
        vbuf[pl.ds(i * 16, 16)] = 1.0 / jnp.maximum(v, 1.0)

    pltpu.sync_copy(vbuf, out.at[pl.ds(base, ROWS_PER_TILE)])


@functools.partial(
    pl.kernel,
    out_type=[
        jax.ShapeDtypeStruct((TP,), jnp.float32),
        jax.ShapeDtypeStruct((TP,), jnp.float32),
    ],
    mesh=_MESH,
    scratch_types=[
        pltpu.VMEM_SHARED((TP,), jnp.float32),
        pltpu.VMEM((PASS_CPT, CH), jnp.int32),
        pltpu.VMEM((CH,), jnp.float32),
        pltpu.VMEM((ROWS_PER_TILE,), jnp.float32),
        pltpu.SemaphoreType.DMA,
        pltpu.SemaphoreType.DMA,
    ],
)
def _deg_kernel(eidx, nidx, invde, invdv, acc, idx_all, ones, vbuf,
                sem_a, sem_b):
    sem = (sem_a, sem_b)
    cid = lax.axis_index("c")

    @pl.when(cid == 0)
    def _():
        _deg_body(eidx, invde, acc, idx_all, ones, vbuf, sem)

    @pl.when(cid == 1)
    def _():
        _deg_body(nidx, invdv, acc, idx_all, ones, vbuf, sem)


def _mm_body(x_ref, w_ref, b_ref, o_ref):
    o_ref[...] = jnp.dot(x_ref[...], w_ref[...],
                         preferred_element_type=jnp.float32) + b_ref[...]


def _matmul_bias(x, w, b):
    m = x.shape[0]
    blk = 1024
    return pl.pallas_call(
        _mm_body,
        grid=(m // blk,),
        in_specs=[
            pl.BlockSpec((blk, D), lambda i: (i, 0)),
            pl.BlockSpec((D, D), lambda i: (0, 0)),
            pl.BlockSpec((1, D), lambda i: (0, 0)),
        ],
        out_specs=pl.BlockSpec((blk, D), lambda i: (i, 0)),
        out_shape=jax.ShapeDtypeStruct((m, D), jnp.float32),
    )(x, w, b)


@jax.jit
def kernel(x, hg, W1, b1, W2, b2):
    nidx = hg[0].astype(jnp.int32)
    eidx = hg[1].astype(jnp.int32)
    pad = jnp.full((NNZP - NNZ,), PAD_ROW, jnp.int32)
    nidx = jnp.concatenate([nidx, pad])
    eidx = jnp.concatenate([eidx, pad])

    invde, invdv = _deg_kernel(eidx.reshape(NNZP // CH, CH),
                               nidx.reshape(NNZP // CH, CH))

    n3 = nidx.reshape(NNZP // ECH, ECH)
    e3 = eidx.reshape(NNZP // ECH, ECH)
    comb_ne = jnp.concatenate([n3, e3], axis=1)
    comb_en = jnp.concatenate([e3, n3], axis=1)

    xp = jnp.pad(x, ((0, TP - N), (0, 0)))
    h = xp
    for (w, b) in ((W1, b1), (W2, b2)):
        xw = _matmul_bias(h, w, b.reshape(1, D))
        e0, e1 = _edge_pass(xw[:, :HC], xw[:, HC:], comb_ne, invde)
        h0, h1 = _node_pass(e0, e1, comb_en, invdv)
        h = jnp.concatenate([h0, h1], axis=1)
    return h[:N]
